# baseline (speedup 1.0000x reference)
import functools

import jax
import jax.numpy as jnp
from jax import lax
from jax.experimental import pallas as pl
from jax.experimental.pallas import tpu as pltpu
from jax.experimental.pallas import tpu_sc as plsc

N_BATCH = 2048
N_REL = 3
DEG = 16
FEAT_DIM = 512
EMBED_DIM = 512

NC = 2
NS = 16
NW = NC * NS
ROWS_PER_NODE = N_REL * DEG
IDX_PER_CHUNK = ROWS_PER_NODE
LANES = 16
COLS = FEAT_DIM // LANES

_SC_MESH = plsc.VectorSubcoreMesh(
    core_axis_name="c", subcore_axis_name="s", num_cores=NC, num_subcores=NS
)


NODES_PER_CHUNK = 2
CHUNK_IDX = NODES_PER_CHUNK * ROWS_PER_NODE


def _make_sc(n_batch):
    b_per_w = n_batch // NW
    chunks = b_per_w // NODES_PER_CHUNK
    half_nodes = b_per_w // 2
    half_chunks = chunks // 2

    @functools.partial(
        pl.kernel,
        out_type=jax.ShapeDtypeStruct((n_batch, FEAT_DIM), jnp.float32),
        mesh=_SC_MESH,
        scratch_types=[
            pltpu.VMEM((b_per_w,), jnp.int32),
            pltpu.VMEM((chunks, CHUNK_IDX), jnp.int32),
            pltpu.VMEM((CHUNK_IDX, FEAT_DIM), jnp.float32),
            pltpu.VMEM((CHUNK_IDX, FEAT_DIM), jnp.float32),
            pltpu.VMEM((half_nodes, FEAT_DIM), jnp.float32),
            pltpu.SemaphoreType.DMA,
            pltpu.SemaphoreType.DMA,
            pltpu.SemaphoreType.DMA,
        ],
    )
    def sc_gather_agg(
        feats_hbm, self_idx_hbm, idx_hbm, agg_hbm, sidx_v, idx_v, buf0, buf1,
        stg, ssem, sem0, sem1
    ):
        wid = lax.axis_index("s") * NC + lax.axis_index("c")
        pltpu.sync_copy(self_idx_hbm.at[pl.ds(wid * b_per_w, b_per_w)], sidx_v)
        pltpu.sync_copy(idx_hbm.at[wid], idx_v)

        inv_deg = jnp.float32(1.0 / DEG)
        bufs = ((buf0, sem0), (buf1, sem1))

        def _seed(h):
            pltpu.async_copy(
                feats_hbm.at[sidx_v.at[pl.ds(h * half_nodes, half_nodes)]],
                stg, ssem,
            )

        def _seed_wait():
            pltpu.make_async_copy(
                feats_hbm.at[sidx_v.at[pl.ds(0, half_nodes)]], stg, ssem
            ).wait()

        def _start(ci, b, s):
            pltpu.async_copy(feats_hbm.at[idx_v.at[ci]], b, s)

        def _wait(ci, b, s):
            pltpu.make_async_copy(feats_hbm.at[idx_v.at[ci]], b, s).wait()

        def _accum(ci, buf, half):
            for j in range(NODES_PER_CHUNK):
                base = j * ROWS_PER_NODE

                def acc_body(k, acc):
                    return tuple(
                        acc[c] + buf[base + k, pl.ds(c * LANES, LANES)]
                        for c in range(COLS)
                    )

                acc0 = tuple(
                    jnp.zeros((LANES,), jnp.float32) for _ in range(COLS)
                )
                acc = lax.fori_loop(0, ROWS_PER_NODE, acc_body, acc0)
                slot = NODES_PER_CHUNK * ci + j - half * half_nodes
                for c in range(COLS):
                    sl = pl.ds(c * LANES, LANES)
                    stg[slot, sl] = stg[slot, sl] + acc[c] * inv_deg

        _seed(0)
        _start(0, buf0, sem0)
        _start(1, buf1, sem1)
        _seed_wait()

        def _half_pairs(half):
            def pair_body(i, _):
                c0 = half * half_chunks + 2 * i
                for off, (b, s) in enumerate(bufs):
                    ci = c0 + off
                    _wait(ci, b, s)
                    _accum(ci, b, half)

                    @pl.when(ci + 2 < chunks)
                    def _prefetch():
                        _start(ci + 2, b, s)

                return _

            lax.fori_loop(0, half_chunks // 2, pair_body, None)

        _half_pairs(0)
        pltpu.sync_copy(
            stg, agg_hbm.at[pl.ds(wid * b_per_w, half_nodes)]
        )
        _seed(1)
        _seed_wait()
        _half_pairs(1)
        pltpu.sync_copy(
            stg, agg_hbm.at[pl.ds(wid * b_per_w + half_nodes, half_nodes)]
        )

    return sc_gather_agg


def _mm_body(agg_ref, w_ref, o_ref):
    o_ref[...] = jnp.maximum(
        jnp.dot(agg_ref[...], w_ref[...], preferred_element_type=jnp.float32)
        * 0.25,
        0.0,
    )


def _make_mm(n_batch, grid):
    return pl.pallas_call(
        _mm_body,
        out_shape=jax.ShapeDtypeStruct((n_batch, EMBED_DIM), jnp.float32),
        grid=(grid,),
        in_specs=[
            pl.BlockSpec((n_batch // grid, FEAT_DIM), lambda i: (i, 0)),
            pl.BlockSpec((FEAT_DIM, EMBED_DIM), lambda i: (0, 0)),
        ],
        out_specs=pl.BlockSpec((n_batch // grid, EMBED_DIM), lambda i: (i, 0)),
    )


_sc_full = _make_sc(N_BATCH)
_mm_full = _make_mm(N_BATCH, 8)


@jax.jit
def kernel(features, weight, nodes, neigh_idx):
    nodes = nodes.astype(jnp.int32)
    neigh_idx = neigh_idx.astype(jnp.int32)
    idx_all = neigh_idx.transpose(1, 0, 2).reshape(
        NW, (N_BATCH // NW) // NODES_PER_CHUNK, CHUNK_IDX
    )
    agg = _sc_full(features, nodes, idx_all)
    return _mm_full(agg, weight)


# speedup vs baseline: 1.1095x; 1.1095x over previous
import functools

import jax
import jax.numpy as jnp
from jax import lax
from jax.experimental import pallas as pl
from jax.experimental.pallas import tpu as pltpu
from jax.experimental.pallas import tpu_sc as plsc

N_BATCH = 2048
N_REL = 3
DEG = 16
FEAT_DIM = 512
EMBED_DIM = 512

NC = 2
NS = 16
NW = NC * NS
ROWS_PER_NODE = N_REL * DEG
IDX_PER_CHUNK = ROWS_PER_NODE
LANES = 16
COLS = FEAT_DIM // LANES

_SC_MESH = plsc.VectorSubcoreMesh(
    core_axis_name="c", subcore_axis_name="s", num_cores=NC, num_subcores=NS
)


NBUF = 4


def _make_sc(n_batch):
    b_per_w = n_batch // NW
    chunks = b_per_w
    half_nodes = b_per_w // 2
    half_chunks = chunks // 2

    @functools.partial(
        pl.kernel,
        out_type=jax.ShapeDtypeStruct((n_batch, FEAT_DIM), jnp.float32),
        mesh=_SC_MESH,
        scratch_types=[
            pltpu.VMEM((b_per_w,), jnp.int32),
            pltpu.VMEM((chunks, IDX_PER_CHUNK), jnp.int32),
            [pltpu.VMEM((IDX_PER_CHUNK, FEAT_DIM), jnp.float32)] * NBUF,
            pltpu.VMEM((half_nodes, FEAT_DIM), jnp.float32),
            [pltpu.SemaphoreType.DMA] * NBUF,
            pltpu.SemaphoreType.DMA,
        ],
    )
    def sc_gather_agg(
        feats_hbm, self_idx_hbm, idx_hbm, agg_hbm, sidx_v, idx_v, bufs,
        stg, sems, ssem
    ):
        wid = lax.axis_index("s") * NC + lax.axis_index("c")
        pltpu.sync_copy(self_idx_hbm.at[pl.ds(wid * b_per_w, b_per_w)], sidx_v)
        pltpu.sync_copy(idx_hbm.at[wid], idx_v)

        inv_deg = jnp.float32(1.0 / DEG)

        def _seed(h):
            pltpu.async_copy(
                feats_hbm.at[sidx_v.at[pl.ds(h * half_nodes, half_nodes)]],
                stg, ssem,
            )

        def _seed_wait():
            pltpu.make_async_copy(
                feats_hbm.at[sidx_v.at[pl.ds(0, half_nodes)]], stg, ssem
            ).wait()

        def _start(ci, j):
            pltpu.async_copy(feats_hbm.at[idx_v.at[ci]], bufs[j], sems[j])

        def _wait(ci, j):
            pltpu.make_async_copy(
                feats_hbm.at[idx_v.at[ci]], bufs[j], sems[j]
            ).wait()

        def _accum(ci, j, half):
            buf = bufs[j]

            def acc_body(k, acc):
                return tuple(
                    acc[c] + buf[k, pl.ds(c * LANES, LANES)]
                    for c in range(COLS)
                )

            acc0 = tuple(jnp.zeros((LANES,), jnp.float32) for _ in range(COLS))
            acc = lax.fori_loop(0, ROWS_PER_NODE, acc_body, acc0)
            slot = ci - half * half_nodes
            for c in range(COLS):
                sl = pl.ds(c * LANES, LANES)
                stg[slot, sl] = stg[slot, sl] + acc[c] * inv_deg

        _seed(0)
        for j in range(NBUF):
            _start(j, j)
        _seed_wait()

        def _half_quads(half):
            def quad_body(i, _):
                c0 = half * half_chunks + NBUF * i
                for off in range(NBUF):
                    ci = c0 + off
                    _wait(ci, off)
                    _accum(ci, off, half)

                    @pl.when(ci + NBUF < chunks)
                    def _prefetch():
                        _start(ci + NBUF, off)

                return _

            lax.fori_loop(0, half_chunks // NBUF, quad_body, None)

        _half_quads(0)
        pltpu.sync_copy(stg, agg_hbm.at[pl.ds(wid * b_per_w, half_nodes)])
        _seed(1)
        _seed_wait()
        _half_quads(1)
        pltpu.sync_copy(
            stg, agg_hbm.at[pl.ds(wid * b_per_w + half_nodes, half_nodes)]
        )

    return sc_gather_agg


def _mm_body(agg_ref, w_ref, o_ref):
    o_ref[...] = jnp.maximum(
        jnp.dot(agg_ref[...], w_ref[...], preferred_element_type=jnp.float32)
        * 0.25,
        0.0,
    )


def _make_mm(n_batch, grid):
    return pl.pallas_call(
        _mm_body,
        out_shape=jax.ShapeDtypeStruct((n_batch, EMBED_DIM), jnp.float32),
        grid=(grid,),
        in_specs=[
            pl.BlockSpec((n_batch // grid, FEAT_DIM), lambda i: (i, 0)),
            pl.BlockSpec((FEAT_DIM, EMBED_DIM), lambda i: (0, 0)),
        ],
        out_specs=pl.BlockSpec((n_batch // grid, EMBED_DIM), lambda i: (i, 0)),
    )


_sc_full = _make_sc(N_BATCH)
_mm_full = _make_mm(N_BATCH, 8)


@jax.jit
def kernel(features, weight, nodes, neigh_idx):
    nodes = nodes.astype(jnp.int32)
    neigh_idx = neigh_idx.astype(jnp.int32)
    idx_all = neigh_idx.transpose(1, 0, 2).reshape(
        NW, N_BATCH // NW, IDX_PER_CHUNK
    )
    agg = _sc_full(features, nodes, idx_all)
    return _mm_full(agg, weight)
